# baseline (speedup 1.0000x reference)
import functools

import jax
import jax.numpy as jnp
from jax import lax
from jax.experimental import pallas as pl
from jax.experimental.pallas import tpu as pltpu
from jax.experimental.pallas import tpu_sc as plsc

_VOCAB = 1000000
_CTX = 200
_PATH = 20
_EMB = 64
_SLOTS = 7
_LAST_BLK = (_VOCAB // 128) * 128


def _body(ctx_idx, nid, tbl_t, nod_t, tail_blk, out_hbm,
          idx_v, blk, nblk, nidx_v, acc_v, shared, sums_v, out_v, sem, nsem):
    cid = lax.axis_index("c")
    sid = lax.axis_index("s")
    w = cid * 16 + sid
    lanes = lax.iota(jnp.int32, 16)
    zero = jnp.zeros((16,), jnp.float32)
    base = w * _SLOTS

    @pl.when(sid == 0)
    def _():
        pltpu.async_copy(nod_t.at[:, pl.ds(0, 128)], nblk, nsem)

    base8 = pl.multiple_of((base >> 3) << 3, 8)

    @pl.when(base8 + 16 <= _CTX)
    def _():
        pltpu.sync_copy(ctx_idx.at[pl.ds(base8, 16)], idx_v.at[pl.ds(0, 16)])

    @pl.when((base8 + 16 > _CTX) & (base < _CTX))
    def _():
        pltpu.sync_copy(ctx_idx.at[pl.ds(_CTX - 8, 8)], idx_v.at[pl.ds(0, 8)])

    vec = idx_v[pl.ds(base - base8, 16)]

    def issue(l):
        i = vec[l]
        start = pl.multiple_of((i >> 7) << 7, 128)
        sv = (base + l) < _CTX
        b = blk.at[l]

        @pl.when(sv & (start < _LAST_BLK))
        def _():
            pltpu.async_copy(tbl_t.at[:, pl.ds(start, 128)], b, sem)

        @pl.when(sv & (start >= _LAST_BLK))
        def _():
            pltpu.async_copy(tail_blk, b, sem)

    for l in range(_SLOTS):
        issue(l)

    a0, a1, a2, a3 = zero, zero, zero, zero
    for l in range(_SLOTS):
        sv = (base + l) < _CTX

        @pl.when(sv)
        def _():
            pltpu.make_async_copy(tail_blk, blk.at[l], sem).wait()

        i = vec[l]
        off = (jnp.full((16,), 0, jnp.int32) + (i & 127)
               + jnp.where(i >= _LAST_BLK, 64, 0))
        svalid = (lanes * 0 + base + l) < _CTX
        b = blk.at[l]
        c0 = plsc.load_gather(b, [lanes, off])
        c1 = plsc.load_gather(b, [lanes + 16, off])
        c2 = plsc.load_gather(b, [lanes + 32, off])
        c3 = plsc.load_gather(b, [lanes + 48, off])
        a0 = a0 + jnp.where(svalid, c0, 0.0)
        a1 = a1 + jnp.where(svalid, c1, 0.0)
        a2 = a2 + jnp.where(svalid, c2, 0.0)
        a3 = a3 + jnp.where(svalid, c3, 0.0)

    acc_v[pl.ds(0, 16)] = a0
    acc_v[pl.ds(16, 16)] = a1
    acc_v[pl.ds(32, 16)] = a2
    acc_v[pl.ds(48, 16)] = a3
    pltpu.sync_copy(acc_v, shared.at[sid])

    plsc.subcore_barrier()

    @pl.when(sid == 0)
    def _():
        pltpu.sync_copy(shared, sums_v)
        inv = 1.0 / _CTX
        v = [zero, zero, zero, zero]
        for r in range(16):
            for q in range(4):
                v[q] = v[q] + sums_v[r, pl.ds(16 * q, 16)]
        v = [x * inv for x in v]

        pltpu.sync_copy(nid, nidx_v.at[pl.ds(0, _PATH)])
        pltpu.make_async_copy(nod_t.at[:, pl.ds(0, 128)], nblk, nsem).wait()
        nid0 = nidx_v[pl.ds(0, 16)]
        nid1 = jnp.where(lanes < _PATH - 16, nidx_v[pl.ds(16, 16)], 0)
        lg0, lg1 = zero, zero
        for d in range(_EMB):
            vd = v[d // 16][d % 16]
            dd = jnp.full((16,), d, jnp.int32)
            lg0 = lg0 + plsc.load_gather(nblk, [dd, nid0]) * vd
            lg1 = lg1 + plsc.load_gather(nblk, [dd, nid1]) * vd

        out_v[pl.ds(0, 16)] = lg0
        out_v[pl.ds(16, 16)] = lg1
        pltpu.sync_copy(out_v, out_hbm.at[pl.ds(cid * 32, 32)])


def _tc_body(lg_ref, codes_ref, out_ref):
    lg = lg_ref[pl.ds(0, 32)] + lg_ref[pl.ds(32, 32)]
    cd = jnp.concatenate([codes_ref[...], jnp.zeros((12,), jnp.float32)])
    valid = (lax.iota(jnp.int32, 32) < _PATH).astype(jnp.float32)
    sg = 1.0 / (1.0 + jnp.exp(-lg))
    p = jnp.where(cd == 1.0, sg, 1.0 - sg)
    loss = -jnp.sum(jnp.log(p + 1e-9) * valid)
    out_ref[...] = jnp.full((1,), loss, jnp.float32)


@jax.jit
def _run(ctx_idx, nid, codes, tbl_t, nod_t, tail_blk):
    mesh = plsc.VectorSubcoreMesh(core_axis_name="c", subcore_axis_name="s")
    lg = pl.kernel(
        _body,
        out_type=jax.ShapeDtypeStruct((64,), jnp.float32),
        mesh=mesh,
        compiler_params=pltpu.CompilerParams(
            needs_layout_passes=False, use_tc_tiling_on_sc=True),
        scratch_types=[
            pltpu.VMEM((24,), jnp.int32),
            pltpu.VMEM((_SLOTS, _EMB, 128), jnp.float32),
            pltpu.VMEM((_EMB, 128), jnp.float32),
            pltpu.VMEM((32,), jnp.int32),
            pltpu.VMEM((_EMB,), jnp.float32),
            pltpu.VMEM_SHARED((16, _EMB), jnp.float32),
            pltpu.VMEM((16, _EMB), jnp.float32),
            pltpu.VMEM((32,), jnp.float32),
            pltpu.SemaphoreType.DMA,
            pltpu.SemaphoreType.DMA,
        ],
    )(ctx_idx, nid, tbl_t, nod_t, tail_blk)
    loss = pl.pallas_call(
        _tc_body,
        out_shape=jax.ShapeDtypeStruct((1,), jnp.float32),
    )(lg, codes)
    return loss[0]


def kernel(context_idxs, node_ids, codes, in_embed, node_embed):
    tail = in_embed[_VOCAB - 128:].T
    return _run(context_idxs.astype(jnp.int32), node_ids.astype(jnp.int32),
                codes.astype(jnp.float32), in_embed.T, node_embed.T, tail)


# speedup vs baseline: 1.0049x; 1.0049x over previous
import functools

import jax
import jax.numpy as jnp
from jax import lax
from jax.experimental import pallas as pl
from jax.experimental.pallas import tpu as pltpu
from jax.experimental.pallas import tpu_sc as plsc

_VOCAB = 1000000
_CTX = 200
_PATH = 20
_EMB = 64
_SLOTS = 7
_LAST_BLK = (_VOCAB // 128) * 128


def _body(ctx_idx, nid, tbl_t, nod_t, tail_blk, out_hbm,
          idx_v, blk, nblk, nidx_v, acc_v, shared, sums_v, out_v, sem, nsem):
    cid = lax.axis_index("c")
    sid = lax.axis_index("s")
    w = cid * 16 + sid
    lanes = lax.iota(jnp.int32, 16)
    zero = jnp.zeros((16,), jnp.float32)
    base = w * _SLOTS

    @pl.when(sid == 0)
    def _():
        pltpu.async_copy(nod_t.at[:, pl.ds(0, 128)], nblk, nsem)

    base8 = pl.multiple_of((base >> 3) << 3, 8)

    @pl.when(base8 + 16 <= _CTX)
    def _():
        pltpu.sync_copy(ctx_idx.at[pl.ds(base8, 16)], idx_v.at[pl.ds(0, 16)])

    @pl.when((base8 + 16 > _CTX) & (base < _CTX))
    def _():
        pltpu.sync_copy(ctx_idx.at[pl.ds(_CTX - 8, 8)], idx_v.at[pl.ds(0, 8)])

    vec = idx_v[pl.ds(base - base8, 16)]

    def issue(l):
        i = vec[l]
        start = pl.multiple_of((i >> 7) << 7, 128)
        sv = (base + l) < _CTX
        b = blk.at[l]

        @pl.when(sv & (start < _LAST_BLK))
        def _():
            pltpu.async_copy(tbl_t.at[:, pl.ds(start, 128)], b, sem)

        @pl.when(sv & (start >= _LAST_BLK))
        def _():
            pltpu.async_copy(tail_blk, b, sem)

    for l in range(_SLOTS):
        issue(l)

    a0, a1, a2, a3 = zero, zero, zero, zero
    for l in range(_SLOTS):
        sv = (base + l) < _CTX

        @pl.when(sv)
        def _():
            pltpu.make_async_copy(tail_blk, blk.at[l], sem).wait()

        i = vec[l]
        svalid = (lanes * 0 + base + l) < _CTX
        off = jnp.where(svalid,
                        jnp.full((16,), 0, jnp.int32) + (i & 127)
                        + jnp.where(i >= _LAST_BLK, 64, 0), 0)
        b = blk.at[l]
        c0 = plsc.load_gather(b, [lanes, off])
        c1 = plsc.load_gather(b, [lanes + 16, off])
        c2 = plsc.load_gather(b, [lanes + 32, off])
        c3 = plsc.load_gather(b, [lanes + 48, off])
        a0 = a0 + jnp.where(svalid, c0, 0.0)
        a1 = a1 + jnp.where(svalid, c1, 0.0)
        a2 = a2 + jnp.where(svalid, c2, 0.0)
        a3 = a3 + jnp.where(svalid, c3, 0.0)

    acc_v[pl.ds(0, 16)] = a0
    acc_v[pl.ds(16, 16)] = a1
    acc_v[pl.ds(32, 16)] = a2
    acc_v[pl.ds(48, 16)] = a3
    pltpu.sync_copy(acc_v, shared.at[sid])

    plsc.subcore_barrier()

    @pl.when(sid == 0)
    def _():
        pltpu.sync_copy(shared, sums_v)
        inv = 1.0 / _CTX
        v = [zero, zero, zero, zero]
        for r in range(16):
            for q in range(4):
                v[q] = v[q] + sums_v[r, pl.ds(16 * q, 16)]
        v = [x * inv for x in v]

        pltpu.sync_copy(nid, nidx_v.at[pl.ds(0, _PATH)])
        pltpu.make_async_copy(nod_t.at[:, pl.ds(0, 128)], nblk, nsem).wait()
        nid0 = nidx_v[pl.ds(0, 16)]
        nid1 = jnp.where(lanes < _PATH - 16, nidx_v[pl.ds(16, 16)], 0)
        lg0, lg1 = zero, zero
        for d in range(_EMB):
            vd = v[d // 16][d % 16]
            dd = jnp.full((16,), d, jnp.int32)
            lg0 = lg0 + plsc.load_gather(nblk, [dd, nid0]) * vd
            lg1 = lg1 + plsc.load_gather(nblk, [dd, nid1]) * vd

        out_v[pl.ds(0, 16)] = lg0
        out_v[pl.ds(16, 16)] = lg1
        pltpu.sync_copy(out_v, out_hbm.at[pl.ds(cid * 32, 32)])


def _tc_body(lg_ref, codes_ref, out_ref):
    lg = lg_ref[pl.ds(0, 32)] + lg_ref[pl.ds(32, 32)]
    cd = jnp.concatenate([codes_ref[...], jnp.zeros((12,), jnp.float32)])
    valid = (lax.iota(jnp.int32, 32) < _PATH).astype(jnp.float32)
    sg = 1.0 / (1.0 + jnp.exp(-lg))
    p = jnp.where(cd == 1.0, sg, 1.0 - sg)
    loss = -jnp.sum(jnp.log(p + 1e-9) * valid)
    out_ref[...] = jnp.full((1,), loss, jnp.float32)


@jax.jit
def _run(ctx_idx, nid, codes, tbl_t, nod_t, tail_blk):
    mesh = plsc.VectorSubcoreMesh(core_axis_name="c", subcore_axis_name="s")
    lg = pl.kernel(
        _body,
        out_type=jax.ShapeDtypeStruct((64,), jnp.float32),
        mesh=mesh,
        compiler_params=pltpu.CompilerParams(
            needs_layout_passes=False, use_tc_tiling_on_sc=True),
        scratch_types=[
            pltpu.VMEM((24,), jnp.int32),
            pltpu.VMEM((_SLOTS, _EMB, 128), jnp.float32),
            pltpu.VMEM((_EMB, 128), jnp.float32),
            pltpu.VMEM((32,), jnp.int32),
            pltpu.VMEM((_EMB,), jnp.float32),
            pltpu.VMEM_SHARED((16, _EMB), jnp.float32),
            pltpu.VMEM((16, _EMB), jnp.float32),
            pltpu.VMEM((32,), jnp.float32),
            pltpu.SemaphoreType.DMA,
            pltpu.SemaphoreType.DMA,
        ],
    )(ctx_idx, nid, tbl_t, nod_t, tail_blk)
    loss = pl.pallas_call(
        _tc_body,
        out_shape=jax.ShapeDtypeStruct((1,), jnp.float32),
    )(lg, codes)
    return loss[0]


def kernel(context_idxs, node_ids, codes, in_embed, node_embed):
    tail = in_embed[_VOCAB - 128:].T
    return _run(context_idxs.astype(jnp.int32), node_ids.astype(jnp.int32),
                codes.astype(jnp.float32), in_embed.T, node_embed.T, tail)
